# Initial kernel scaffold; baseline (speedup 1.0000x reference)
import jax
import jax.numpy as jnp
from jax.experimental import pallas as pl


def kernel(hidden_states, gate_w, expert_bias):
    raise NotImplementedError("write your pallas kernel here")


# speedup vs baseline: 3.4965x; 3.4965x over previous
import functools

import jax
import jax.numpy as jnp
from jax.experimental import pallas as pl

_T = 32768
_E = 8
_K = 2
_BLK = 1024


def _router_kernel(h_ref, gw_ref, bias_ref, probs_ref, idx_ref, map_ref):
    h = h_ref[...]
    gw = gw_ref[...]
    logits = jax.lax.dot_general(
        h.astype(jnp.bfloat16), gw.astype(jnp.bfloat16),
        (((1,), (1,)), ((), ())),
        preferred_element_type=jnp.float32,
    )
    m = jnp.max(logits, axis=-1, keepdims=True)
    ex = jnp.exp(logits - m)
    scores = ex / jnp.sum(ex, axis=-1, keepdims=True)
    sel = scores + bias_ref[...]

    eidx = jax.lax.broadcasted_iota(jnp.int32, sel.shape, 1)
    m1 = jnp.max(sel, axis=-1, keepdims=True)
    i1 = jnp.min(jnp.where(sel == m1, eidx, _E), axis=-1, keepdims=True)
    sel2 = jnp.where(eidx == i1, -jnp.inf, sel)
    m2 = jnp.max(sel2, axis=-1, keepdims=True)
    i2 = jnp.min(jnp.where(sel2 == m2, eidx, _E), axis=-1, keepdims=True)

    one1 = eidx == i1
    one2 = eidx == i2
    p1 = jnp.sum(jnp.where(one1, scores, 0.0), axis=-1, keepdims=True)
    p2 = jnp.sum(jnp.where(one2, scores, 0.0), axis=-1, keepdims=True)
    denom = p1 + p2 + 1e-9
    probs_ref[...] = jnp.concatenate([p1 / denom, p2 / denom], axis=1)
    idx_ref[...] = jnp.concatenate([i1, i2], axis=1)
    map_ref[...] = (one1 | one2).astype(jnp.int8)


@jax.jit
def kernel(hidden_states, gate_w, expert_bias):
    t = hidden_states.shape[0]
    e = gate_w.shape[0]
    bias2d = expert_bias.reshape(1, e)
    grid = t // _BLK
    probs, idx, rmap = pl.pallas_call(
        _router_kernel,
        grid=(grid,),
        in_specs=[
            pl.BlockSpec((_BLK, hidden_states.shape[1]), lambda i: (i, 0)),
            pl.BlockSpec((e, hidden_states.shape[1]), lambda i: (0, 0)),
            pl.BlockSpec((1, e), lambda i: (0, 0)),
        ],
        out_specs=[
            pl.BlockSpec((_BLK, _K), lambda i: (i, 0)),
            pl.BlockSpec((_BLK, _K), lambda i: (i, 0)),
            pl.BlockSpec((_BLK, e), lambda i: (i, 0)),
        ],
        out_shape=[
            jax.ShapeDtypeStruct((t, _K), jnp.float32),
            jax.ShapeDtypeStruct((t, _K), jnp.int32),
            jax.ShapeDtypeStruct((t, e), jnp.int8),
        ],
    )(hidden_states, gate_w, bias2d)
    aux_loss = jnp.zeros((), dtype=jnp.float32)
    return probs, idx, rmap.astype(jnp.bool_), aux_loss


# speedup vs baseline: 5.6810x; 1.6248x over previous
import jax
import jax.numpy as jnp
from jax.experimental import pallas as pl
from jax.experimental.pallas import tpu as pltpu

_T = 32768
_E = 8
_K = 2
_BLK = 1024


def _router_kernel(h_ref, gw_ref, bias_ref, probs_ref, idx_ref, map_ref):
    h = h_ref[...]
    gw = gw_ref[...]
    logits = jax.lax.dot_general(
        h.astype(jnp.bfloat16), gw.astype(jnp.bfloat16),
        (((1,), (1,)), ((), ())),
        preferred_element_type=jnp.float32,
    )
    lt = jax.lax.transpose(logits, (1, 0))
    m = jnp.max(lt, axis=0, keepdims=True)
    ex = jnp.exp(lt - m)
    scores = ex / jnp.sum(ex, axis=0, keepdims=True)
    sel = scores + bias_ref[...]

    eidx = jax.lax.broadcasted_iota(jnp.int32, sel.shape, 0)
    m1 = jnp.max(sel, axis=0, keepdims=True)
    i1 = jnp.min(jnp.where(sel == m1, eidx, _E), axis=0, keepdims=True)
    sel2 = jnp.where(eidx == i1, -jnp.inf, sel)
    m2 = jnp.max(sel2, axis=0, keepdims=True)
    i2 = jnp.min(jnp.where(sel2 == m2, eidx, _E), axis=0, keepdims=True)

    one1 = eidx == i1
    one2 = eidx == i2
    p1 = jnp.sum(jnp.where(one1, scores, 0.0), axis=0, keepdims=True)
    p2 = jnp.sum(jnp.where(one2, scores, 0.0), axis=0, keepdims=True)
    denom = p1 + p2 + 1e-9
    probs_ref[...] = jnp.concatenate([p1 / denom, p2 / denom], axis=0)
    idx_ref[...] = jnp.concatenate([i1, i2], axis=0)
    map_ref[...] = (one1 | one2).astype(jnp.int8)


@jax.jit
def kernel(hidden_states, gate_w, expert_bias):
    t = hidden_states.shape[0]
    e = gate_w.shape[0]
    bias2d = expert_bias.reshape(e, 1)
    grid = t // _BLK
    probs_t, idx_t, rmap_t = pl.pallas_call(
        _router_kernel,
        grid=(grid,),
        in_specs=[
            pl.BlockSpec((_BLK, hidden_states.shape[1]), lambda i: (i, 0)),
            pl.BlockSpec((e, hidden_states.shape[1]), lambda i: (0, 0)),
            pl.BlockSpec((e, 1), lambda i: (0, 0)),
        ],
        out_specs=[
            pl.BlockSpec((_K, _BLK), lambda i: (0, i)),
            pl.BlockSpec((_K, _BLK), lambda i: (0, i)),
            pl.BlockSpec((e, _BLK), lambda i: (0, i)),
        ],
        out_shape=[
            jax.ShapeDtypeStruct((_K, t), jnp.float32),
            jax.ShapeDtypeStruct((_K, t), jnp.int32),
            jax.ShapeDtypeStruct((e, t), jnp.int8),
        ],
        compiler_params=pltpu.CompilerParams(
            dimension_semantics=("arbitrary",),
        ),
    )(hidden_states, gate_w, bias2d)
    probs = probs_t.T
    idx = idx_t.T
    rmap = rmap_t.T.astype(jnp.bool_)
    aux_loss = jnp.zeros((), dtype=jnp.float32)
    return probs, idx, rmap, aux_loss
